# baseline (speedup 1.0000x reference)
import functools

import jax
import jax.numpy as jnp
from jax import lax
from jax.experimental import pallas as pl
from jax.experimental.pallas import tpu as pltpu
from jax.experimental.pallas import tpu_sc as plsc

GRID = 128
NOBJ = 100
NOBJ_P = 128
ANCHOR_W = 4.0
ANCHOR_H = 4.0
EMPTY_WEIGHT = 5.0


def _prep_body(tb_ref, idx_ref, tv_ref):
    tb = tb_ref[...]
    g = jnp.float32(GRID)
    t_x = tb[..., 0] * g
    t_y = tb[..., 1] * g
    gif = jnp.floor(t_x)
    gjf = jnp.floor(t_y)
    gii = gif.astype(jnp.int32)
    gjj = gjf.astype(jnp.int32)
    bsz = tb.shape[0]
    o = lax.broadcasted_iota(jnp.int32, (bsz, NOBJ_P), 1)
    b = lax.broadcasted_iota(jnp.int32, (bsz, NOBJ_P), 0)
    validm = o < NOBJ
    cell = jnp.where(validm, gjj * GRID + gii, -1 - o)
    eq = cell[:, :, None] == cell[:, None, :]
    later = (lax.broadcasted_iota(jnp.int32, (bsz, NOBJ_P, NOBJ_P), 2)
             > lax.broadcasted_iota(jnp.int32, (bsz, NOBJ_P, NOBJ_P), 1))
    dup = jnp.any(eq & later, axis=2)
    winner = jnp.where(validm & ~dup, 1.0, 0.0).astype(jnp.float32)

    idx_ref[...] = jnp.where(validm, b * (GRID * GRID) + cell, 0)

    tx = t_x - gif
    ty = t_y - gjf
    tz = tb[..., 2]
    tw = jnp.log(tb[..., 3] * g / ANCHOR_W + 1e-16)
    th = jnp.log(tb[..., 4] * g / ANCHOR_H + 1e-16)
    td = jnp.log(tb[..., 5] + 1e-16)
    sub = lax.bitwise_and(gii, 7).astype(jnp.float32)
    zero = jnp.zeros_like(tx)
    tv_ref[...] = jnp.stack(
        [tx, ty, tz, tw, th, td,
         t_x, t_y, tb[..., 3] * g, tb[..., 4] * g,
         gif, gjf, winner, sub, zero, zero], axis=-1)


def _run_prep(tb_pad):
    bsz = tb_pad.shape[0]
    return pl.pallas_call(
        _prep_body,
        out_shape=(
            jax.ShapeDtypeStruct((bsz, NOBJ_P), jnp.int32),
            jax.ShapeDtypeStruct((bsz, NOBJ_P, 16), jnp.float32),
        ),
    )(tb_pad)


def _sc_gather_body(b_per_w,
                    idx_hbm, kp_hbm, okp,
                    flat_v, kp_t, sem):
    wid = lax.axis_index("s") * 2 + lax.axis_index("c")
    base = wid * b_per_w
    pltpu.sync_copy(idx_hbm.at[pl.ds(base, b_per_w)], flat_v)
    pltpu.async_copy(kp_hbm.at[flat_v], kp_t, sem).wait()
    pltpu.sync_copy(kp_t, okp.at[pl.ds(base, b_per_w)])


def _run_gather(flat_idx, kp_tab):
    n = flat_idx.shape[0]
    d = kp_tab.shape[1]
    info = plsc.get_sparse_core_info()
    nw = info.num_cores * info.num_subcores
    b_per_w = n // nw
    mesh = plsc.VectorSubcoreMesh(core_axis_name="c", subcore_axis_name="s")
    kern = functools.partial(
        pl.kernel,
        mesh=mesh,
        out_type=jax.ShapeDtypeStruct((n, d), jnp.float32),
        scratch_types=[
            pltpu.VMEM((b_per_w,), jnp.int32),
            pltpu.VMEM((b_per_w, d), jnp.float32),
            pltpu.SemaphoreType.DMA,
        ],
    )(functools.partial(_sc_gather_body, b_per_w))
    return kern(flat_idx, kp_tab)


def _onehot_select(W_r, W_c, slab):
    rows = jnp.dot(W_r, slab, preferred_element_type=jnp.float32)
    return jnp.sum(rows * W_c, axis=1)


def _loss_body(nsteps, ntot,
               x_ref, y_ref, z_ref, wb_ref, hb_ref, d_ref, conf_ref,
               gkp_ref, tkp_ref, tv_blk_ref, tv_full_ref, out_ref, acc_ref):
    i = pl.program_id(0)

    @pl.when(i == 0)
    def _init():
        for k in range(6):
            acc_ref[k] = 0.0

    sp = jax.nn.softplus
    tvb = tv_blk_ref[...]
    nobj = tvb.shape[0]
    gif = tvb[:, 10]
    gjf = tvb[:, 11]
    w = tvb[:, 12]
    sub = tvb[:, 13]

    io_r = lax.broadcasted_iota(jnp.int32, (nobj, GRID), 1)
    W_r = jnp.where(gjf.astype(jnp.int32)[:, None] == io_r, 1.0, 0.0)
    W_c = jnp.where(gif.astype(jnp.int32)[:, None] == io_r, 1.0, 0.0)

    conf_slab = conf_ref[0]
    selx = _onehot_select(W_r, W_c, x_ref[0])
    sely = _onehot_select(W_r, W_c, y_ref[0])
    selz = _onehot_select(W_r, W_c, z_ref[0])
    selw = _onehot_select(W_r, W_c, wb_ref[0])
    selh = _onehot_select(W_r, W_c, hb_ref[0])
    seld = _onehot_select(W_r, W_c, d_ref[0])
    selc = _onehot_select(W_r, W_c, conf_slab)

    nkey = tkp_ref.shape[1]
    sel_kp = gkp_ref[...][:, :nkey]

    d_box = (jnp.abs(selx - tvb[:, 0]) + jnp.abs(sely - tvb[:, 1])
             + jnp.abs(selw - tvb[:, 3]) + jnp.abs(selh - tvb[:, 4])
             + jnp.abs(selz - tvb[:, 2]) + jnp.abs(seld - tvb[:, 5]))
    s_box = jnp.sum(d_box * w)
    s_kp = jnp.sum(jnp.sum(jnp.abs(sel_kp - tkp_ref[...]), axis=1) * w)
    s_n = jnp.sum(w)
    s_corr = jnp.sum(w * (EMPTY_WEIGHT * sp(-selc) - sp(selc)))
    s_base = jnp.sum(sp(conf_slab))

    px = selx + gif
    py = sely + gjf
    pw = jnp.exp(selw) * ANCHOR_W
    ph = jnp.exp(selh) * ANCHOR_H
    vP = w > 0
    px1 = jnp.where(vP, px - 0.5 * pw, 0.0)[:, None]
    px2 = jnp.where(vP, px + 0.5 * pw, 1.0)[:, None]
    py1 = jnp.where(vP, py - 0.5 * ph, 0.0)[:, None]
    py2 = jnp.where(vP, py + 0.5 * ph, 1.0)[:, None]
    areap = jnp.where(vP, pw * ph, 1.0)[:, None]
    wP = w[:, None]

    tvf = tv_full_ref[...]
    wT = tvf[:, 12]
    vT = wT > 0
    ttx, tty, ttw, tth = tvf[:, 6], tvf[:, 7], tvf[:, 8], tvf[:, 9]
    tx1 = jnp.where(vT, ttx - 0.5 * ttw, 0.0)[None, :]
    tx2 = jnp.where(vT, ttx + 0.5 * ttw, 1.0)[None, :]
    ty1 = jnp.where(vT, tty - 0.5 * tth, 0.0)[None, :]
    ty2 = jnp.where(vT, tty + 0.5 * tth, 1.0)[None, :]
    areat = jnp.where(vT, ttw * tth, 1.0)[None, :]
    wTr = wT[None, :]

    ix = jnp.maximum(jnp.minimum(px2, tx2) - jnp.maximum(px1, tx1), 0.0)
    iy = jnp.maximum(jnp.minimum(py2, ty2) - jnp.maximum(py1, ty1), 0.0)
    inter = ix * iy
    union = areap + areat - inter
    s_iou = jnp.sum((union - inter) / union * (wP * wTr))

    acc_ref[0] += s_box
    acc_ref[1] += s_kp
    acc_ref[2] += s_corr
    acc_ref[3] += s_base
    acc_ref[4] += s_iou
    acc_ref[5] += s_n

    @pl.when(i == nsteps - 1)
    def _fin():
        n = acc_ref[5]
        out_ref[...] = jnp.stack([
            acc_ref[0] / n,
            acc_ref[1] / n,
            (acc_ref[3] + acc_ref[2]) / jnp.float32(ntot),
            acc_ref[4] / n,
        ])


def _run_losses(slabs, conf3, gkp, tkp2d, tv2d):
    bsz, g, _ = conf3.shape
    nent = tv2d.shape[0]
    nkey = tkp2d.shape[1]
    nsteps = bsz
    ntot = bsz * g * g
    img = pl.BlockSpec((1, g, g), lambda i: (i, 0, 0))
    row = lambda i: (i, 0)
    return pl.pallas_call(
        functools.partial(_loss_body, nsteps, ntot),
        grid=(nsteps,),
        in_specs=[img] * 7 + [
            pl.BlockSpec((NOBJ_P, 128), row),
            pl.BlockSpec((NOBJ_P, nkey), row),
            pl.BlockSpec((NOBJ_P, 16), row),
            pl.BlockSpec((nent, 16), lambda i: (0, 0)),
        ],
        out_specs=pl.BlockSpec((4,), lambda i: (0,)),
        out_shape=jax.ShapeDtypeStruct((4,), jnp.float32),
        scratch_shapes=[pltpu.SMEM((8,), jnp.float32)],
    )(*slabs, conf3, gkp, tkp2d, tv2d, tv2d)


def kernel(pred_boxes, pred_confidence, pred_keypoints, target_boxes, target_keypoints):
    bsz, g = pred_boxes.shape[0], pred_boxes.shape[1]
    nkey = pred_keypoints.shape[-1]

    tb_pad = jnp.pad(target_boxes, ((0, 0), (0, NOBJ_P - NOBJ), (0, 0)))
    idx2d, tv = _run_prep(tb_pad)

    flat_idx = idx2d.reshape(bsz * NOBJ_P)
    kp_tab = jnp.pad(pred_keypoints.reshape(bsz * g * g, nkey),
                     ((0, 0), (0, 128 - nkey)))
    gkp = _run_gather(flat_idx, kp_tab)

    slabs = [pred_boxes[..., c] for c in range(6)]
    conf3 = pred_confidence.reshape(bsz, g, g)

    tkp_pad = jnp.pad(target_keypoints, ((0, 0), (0, NOBJ_P - NOBJ), (0, 0)))
    tkp2d = tkp_pad.reshape(bsz * NOBJ_P, nkey)
    tv2d = tv.reshape(bsz * NOBJ_P, 16)
    return _run_losses(slabs, conf3, gkp, tkp2d, tv2d)


# speedup vs baseline: 1.9378x; 1.9378x over previous
import functools

import jax
import jax.numpy as jnp
from jax import lax
from jax.experimental import pallas as pl
from jax.experimental.pallas import tpu as pltpu

GRID = 128
NOBJ = 100
NOBJ_P = 128
ANCHOR_W = 4.0
ANCHOR_H = 4.0
EMPTY_WEIGHT = 5.0


def _prep_body(tb_ref, tv_ref):
    tb = tb_ref[...]
    g = jnp.float32(GRID)
    t_x = tb[..., 0] * g
    t_y = tb[..., 1] * g
    gif = jnp.floor(t_x)
    gjf = jnp.floor(t_y)
    gii = gif.astype(jnp.int32)
    gjj = gjf.astype(jnp.int32)
    bsz = tb.shape[0]
    o = lax.broadcasted_iota(jnp.int32, (bsz, NOBJ_P), 1)
    validm = o < NOBJ
    cell = jnp.where(validm, gjj * GRID + gii, -1 - o)
    eq = cell[:, :, None] == cell[:, None, :]
    later = (lax.broadcasted_iota(jnp.int32, (bsz, NOBJ_P, NOBJ_P), 2)
             > lax.broadcasted_iota(jnp.int32, (bsz, NOBJ_P, NOBJ_P), 1))
    dup = jnp.any(eq & later, axis=2)
    winner = jnp.where(validm & ~dup, 1.0, 0.0).astype(jnp.float32)

    tx = t_x - gif
    ty = t_y - gjf
    tz = tb[..., 2]
    tw = jnp.log(tb[..., 3] * g / ANCHOR_W + 1e-16)
    th = jnp.log(tb[..., 4] * g / ANCHOR_H + 1e-16)
    td = jnp.log(tb[..., 5] + 1e-16)
    zero = jnp.zeros_like(tx)
    tv_ref[...] = jnp.stack(
        [tx, ty, tz, tw, th, td,
         t_x, t_y, tb[..., 3] * g, tb[..., 4] * g,
         gif, gjf, winner, zero, zero, zero], axis=-1)


def _run_prep(tb_pad):
    bsz = tb_pad.shape[0]
    return pl.pallas_call(
        _prep_body,
        out_shape=jax.ShapeDtypeStruct((bsz, NOBJ_P, 16), jnp.float32),
    )(tb_pad)


def _planar_select(slab2d, W_cT, W_rT, nch):
    A = jnp.dot(slab2d, W_cT, preferred_element_type=jnp.float32)
    A = A.reshape(nch, GRID, W_cT.shape[1])
    return jnp.sum(A * W_rT[None], axis=1)


def _loss_body(nsteps, ntot,
               box_ref, conf_ref, kp_ref, tkp_ref,
               tv_blk_ref, tv_full_ref, out_ref, acc_ref):
    i = pl.program_id(0)

    @pl.when(i == 0)
    def _init():
        for k in range(6):
            acc_ref[k] = 0.0

    sp = jax.nn.softplus
    tvb = tv_blk_ref[...]
    nobj = tvb.shape[0]
    gif = tvb[:, 10]
    gjf = tvb[:, 11]
    w = tvb[:, 12]

    io = lax.broadcasted_iota(jnp.int32, (GRID, nobj), 0)
    W_cT = jnp.where(gif.astype(jnp.int32)[None, :] == io, 1.0, 0.0)
    W_rT = jnp.where(gjf.astype(jnp.int32)[None, :] == io, 1.0, 0.0)

    nkey = tkp_ref.shape[1]
    box_sel = _planar_select(box_ref[0].reshape(6 * GRID, GRID), W_cT, W_rT, 6)
    conf_slab = conf_ref[0]
    selc = _planar_select(conf_slab, W_cT, W_rT, 1)[0]
    kpT_sel = _planar_select(kp_ref[0], W_cT, W_rT, nkey)

    selx, sely, selz = box_sel[0], box_sel[1], box_sel[2]
    selw, selh, seld = box_sel[3], box_sel[4], box_sel[5]

    d_box = (jnp.abs(selx - tvb[:, 0]) + jnp.abs(sely - tvb[:, 1])
             + jnp.abs(selw - tvb[:, 3]) + jnp.abs(selh - tvb[:, 4])
             + jnp.abs(selz - tvb[:, 2]) + jnp.abs(seld - tvb[:, 5]))
    s_box = jnp.sum(d_box * w)
    s_kp = jnp.sum(jnp.sum(jnp.abs(kpT_sel - tkp_ref[0]), axis=0) * w)
    s_n = jnp.sum(w)
    s_corr = jnp.sum(w * (EMPTY_WEIGHT * sp(-selc) - sp(selc)))
    s_base = jnp.sum(sp(conf_slab))

    px = selx + gif
    py = sely + gjf
    pw = jnp.exp(selw) * ANCHOR_W
    ph = jnp.exp(selh) * ANCHOR_H
    vP = w > 0
    px1 = jnp.where(vP, px - 0.5 * pw, 0.0)[:, None]
    px2 = jnp.where(vP, px + 0.5 * pw, 1.0)[:, None]
    py1 = jnp.where(vP, py - 0.5 * ph, 0.0)[:, None]
    py2 = jnp.where(vP, py + 0.5 * ph, 1.0)[:, None]
    areap = jnp.where(vP, pw * ph, 1.0)[:, None]
    wP = w[:, None]

    tvf = tv_full_ref[...]
    wT = tvf[:, 12]
    vT = wT > 0
    ttx, tty, ttw, tth = tvf[:, 6], tvf[:, 7], tvf[:, 8], tvf[:, 9]
    tx1 = jnp.where(vT, ttx - 0.5 * ttw, 0.0)[None, :]
    tx2 = jnp.where(vT, ttx + 0.5 * ttw, 1.0)[None, :]
    ty1 = jnp.where(vT, tty - 0.5 * tth, 0.0)[None, :]
    ty2 = jnp.where(vT, tty + 0.5 * tth, 1.0)[None, :]
    areat = jnp.where(vT, ttw * tth, 1.0)[None, :]
    wTr = wT[None, :]

    ix = jnp.maximum(jnp.minimum(px2, tx2) - jnp.maximum(px1, tx1), 0.0)
    iy = jnp.maximum(jnp.minimum(py2, ty2) - jnp.maximum(py1, ty1), 0.0)
    inter = ix * iy
    union = areap + areat - inter
    s_iou = jnp.sum((union - inter) / union * (wP * wTr))

    acc_ref[0] += s_box
    acc_ref[1] += s_kp
    acc_ref[2] += s_corr
    acc_ref[3] += s_base
    acc_ref[4] += s_iou
    acc_ref[5] += s_n

    @pl.when(i == nsteps - 1)
    def _fin():
        n = acc_ref[5]
        out_ref[...] = jnp.stack([
            acc_ref[0] / n,
            acc_ref[1] / n,
            (acc_ref[3] + acc_ref[2]) / jnp.float32(ntot),
            acc_ref[4] / n,
        ])


def _run_losses(box_pl, conf_pl, kp_pl, tkpT, tv):
    bsz, g = conf_pl.shape[0], conf_pl.shape[1]
    nkey = tkpT.shape[1]
    nent = bsz * NOBJ_P
    tv2d = tv.reshape(nent, 16)
    nsteps = bsz
    ntot = bsz * g * g
    return pl.pallas_call(
        functools.partial(_loss_body, nsteps, ntot),
        grid=(nsteps,),
        in_specs=[
            pl.BlockSpec((1, 6, g, g), lambda i: (i, 0, 0, 0)),
            pl.BlockSpec((1, g, g), lambda i: (i, 0, 0)),
            pl.BlockSpec((1, nkey * g, g), lambda i: (i, 0, 0)),
            pl.BlockSpec((1, nkey, NOBJ_P), lambda i: (i, 0, 0)),
            pl.BlockSpec((NOBJ_P, 16), lambda i: (i, 0)),
            pl.BlockSpec((nent, 16), lambda i: (0, 0)),
        ],
        out_specs=pl.BlockSpec((4,), lambda i: (0,)),
        out_shape=jax.ShapeDtypeStruct((4,), jnp.float32),
        scratch_shapes=[pltpu.SMEM((8,), jnp.float32)],
    )(box_pl, conf_pl, kp_pl, tkpT, tv2d, tv2d)


def kernel(pred_boxes, pred_confidence, pred_keypoints, target_boxes, target_keypoints):
    bsz, g = pred_boxes.shape[0], pred_boxes.shape[1]
    nkey = pred_keypoints.shape[-1]

    tb_pad = jnp.pad(target_boxes, ((0, 0), (0, NOBJ_P - NOBJ), (0, 0)))
    tv = _run_prep(tb_pad)

    box_pl = jnp.moveaxis(pred_boxes, 3, 1)
    conf_pl = pred_confidence.reshape(bsz, g, g)
    kp_pl = jnp.moveaxis(pred_keypoints, 3, 1).reshape(bsz, nkey * g, g)

    tkp_pad = jnp.pad(target_keypoints, ((0, 0), (0, NOBJ_P - NOBJ), (0, 0)))
    tkpT = jnp.transpose(tkp_pad, (0, 2, 1))
    return _run_losses(box_pl, conf_pl, kp_pl, tkpT, tv)


# speedup vs baseline: 5.7002x; 2.9417x over previous
import functools

import jax
import jax.numpy as jnp
from jax import lax
from jax.experimental import pallas as pl
from jax.experimental.pallas import tpu as pltpu

GRID = 128
NOBJ = 100
NOBJ_P = 128
ANCHOR_W = 4.0
ANCHOR_H = 4.0
EMPTY_WEIGHT = 5.0


def _prep_body(tb_ref, tv_ref):
    tb = tb_ref[...]
    g = jnp.float32(GRID)
    t_x = tb[..., 0] * g
    t_y = tb[..., 1] * g
    t_w = tb[..., 3] * g
    t_h = tb[..., 4] * g
    gif = jnp.floor(t_x)
    gjf = jnp.floor(t_y)
    gii = gif.astype(jnp.int32)
    gjj = gjf.astype(jnp.int32)
    bsz = tb.shape[0]
    o = lax.broadcasted_iota(jnp.int32, (bsz, NOBJ_P), 1)
    validm = o < NOBJ
    cell = jnp.where(validm, gjj * GRID + gii, -1 - o)
    eq = cell[:, :, None] == cell[:, None, :]
    later = (lax.broadcasted_iota(jnp.int32, (bsz, NOBJ_P, NOBJ_P), 2)
             > lax.broadcasted_iota(jnp.int32, (bsz, NOBJ_P, NOBJ_P), 1))
    dup = jnp.any(eq & later, axis=2)
    win = jnp.where(validm & ~dup, 1.0, 0.0).astype(jnp.float32)

    tx = t_x - gif
    ty = t_y - gjf
    tz = tb[..., 2]
    tw = jnp.log(t_w / ANCHOR_W + 1e-16)
    th = jnp.log(t_h / ANCHOR_H + 1e-16)
    td = jnp.log(tb[..., 5] + 1e-16)
    v = win > 0
    tx1 = jnp.where(v, t_x - 0.5 * t_w, 0.0)
    tx2 = jnp.where(v, t_x + 0.5 * t_w, 1.0)
    ty1 = jnp.where(v, t_y - 0.5 * t_h, 0.0)
    ty2 = jnp.where(v, t_y + 0.5 * t_h, 1.0)
    areat = jnp.where(v, t_w * t_h, 1.0)
    zero = jnp.zeros_like(tx)
    tv_ref[...] = jnp.stack(
        [tx, ty, tz, tw, th, td, win, gif, gjf,
         tx1, tx2, ty1, ty2, areat, zero, zero], axis=1)


def _run_prep(tb_pad):
    bsz = tb_pad.shape[0]
    return pl.pallas_call(
        _prep_body,
        out_shape=jax.ShapeDtypeStruct((bsz, 16, NOBJ_P), jnp.float32),
    )(tb_pad)


def _planar_select(slab2d, W_cT, W_rT, nch):
    A = jnp.dot(slab2d, W_cT, preferred_element_type=jnp.float32)
    A = A.reshape(nch, GRID, W_cT.shape[1])
    return jnp.sum(A * W_rT[None], axis=1)


def _loss_body(nsteps, ntot,
               box_ref, conf_ref, kp_ref, tkp_ref,
               tv_blk_ref, tv_full_ref, out_ref, acc_ref):
    i = pl.program_id(0)

    @pl.when(i == 0)
    def _init():
        for k in range(6):
            acc_ref[k] = 0.0

    sp = jax.nn.softplus
    tvb = tv_blk_ref[0]
    nobj = tvb.shape[1]
    w = tvb[6]
    gif = tvb[7]
    gjf = tvb[8]

    io = lax.broadcasted_iota(jnp.int32, (GRID, nobj), 0)
    W_cT = jnp.where(gif.astype(jnp.int32)[None, :] == io, 1.0, 0.0)
    W_rT = jnp.where(gjf.astype(jnp.int32)[None, :] == io, 1.0, 0.0)

    nkey = tkp_ref.shape[1]
    box_sel = _planar_select(box_ref[0].reshape(6 * GRID, GRID), W_cT, W_rT, 6)
    conf_slab = conf_ref[0]
    selc = _planar_select(conf_slab, W_cT, W_rT, 1)[0]
    kpT_sel = _planar_select(kp_ref[0], W_cT, W_rT, nkey)

    selx, sely, selz = box_sel[0], box_sel[1], box_sel[2]
    selw, selh, seld = box_sel[3], box_sel[4], box_sel[5]

    d_box = (jnp.abs(selx - tvb[0]) + jnp.abs(sely - tvb[1])
             + jnp.abs(selw - tvb[3]) + jnp.abs(selh - tvb[4])
             + jnp.abs(selz - tvb[2]) + jnp.abs(seld - tvb[5]))
    s_box = jnp.sum(d_box * w)
    s_kp = jnp.sum(jnp.sum(jnp.abs(kpT_sel - tkp_ref[0]), axis=0) * w)
    s_n = jnp.sum(w)
    s_corr = jnp.sum(w * (EMPTY_WEIGHT * sp(-selc) - sp(selc)))
    s_base = jnp.sum(sp(conf_slab))

    px = selx + gif
    py = sely + gjf
    pw = jnp.exp(selw) * ANCHOR_W
    ph = jnp.exp(selh) * ANCHOR_H
    vP = w > 0
    px1 = jnp.where(vP, px - 0.5 * pw, 0.0)[:, None]
    px2 = jnp.where(vP, px + 0.5 * pw, 1.0)[:, None]
    py1 = jnp.where(vP, py - 0.5 * ph, 0.0)[:, None]
    py2 = jnp.where(vP, py + 0.5 * ph, 1.0)[:, None]
    areap = jnp.where(vP, pw * ph, 1.0)[:, None]
    wP = w[:, None]

    tvf = tv_full_ref[...]
    tx1 = tvf[9][None, :]
    tx2 = tvf[10][None, :]
    ty1 = tvf[11][None, :]
    ty2 = tvf[12][None, :]
    areat = tvf[13][None, :]
    wTr = tvf[6][None, :]

    ix = jnp.maximum(jnp.minimum(px2, tx2) - jnp.maximum(px1, tx1), 0.0)
    iy = jnp.maximum(jnp.minimum(py2, ty2) - jnp.maximum(py1, ty1), 0.0)
    inter = ix * iy
    union = areap + areat - inter
    s_iou = jnp.sum((union - inter) / union * (wP * wTr))

    acc_ref[0] += s_box
    acc_ref[1] += s_kp
    acc_ref[2] += s_corr
    acc_ref[3] += s_base
    acc_ref[4] += s_iou
    acc_ref[5] += s_n

    @pl.when(i == nsteps - 1)
    def _fin():
        n = acc_ref[5]
        out_ref[...] = jnp.stack([
            acc_ref[0] / n,
            acc_ref[1] / n,
            (acc_ref[3] + acc_ref[2]) / jnp.float32(ntot),
            acc_ref[4] / n,
        ])


def _run_losses(box_pl, conf_pl, kp_pl, tkpT, tvT, tv_full):
    bsz, g = conf_pl.shape[0], conf_pl.shape[1]
    nkey = tkpT.shape[1]
    nent = tv_full.shape[1]
    nsteps = bsz
    ntot = bsz * g * g
    return pl.pallas_call(
        functools.partial(_loss_body, nsteps, ntot),
        grid=(nsteps,),
        in_specs=[
            pl.BlockSpec((1, 6, g, g), lambda i: (i, 0, 0, 0)),
            pl.BlockSpec((1, g, g), lambda i: (i, 0, 0)),
            pl.BlockSpec((1, nkey * g, g), lambda i: (i, 0, 0)),
            pl.BlockSpec((1, nkey, NOBJ_P), lambda i: (i, 0, 0)),
            pl.BlockSpec((1, 16, NOBJ_P), lambda i: (i, 0, 0)),
            pl.BlockSpec((16, nent), lambda i: (0, 0)),
        ],
        out_specs=pl.BlockSpec((4,), lambda i: (0,)),
        out_shape=jax.ShapeDtypeStruct((4,), jnp.float32),
        scratch_shapes=[pltpu.SMEM((8,), jnp.float32)],
    )(box_pl, conf_pl, kp_pl, tkpT, tvT, tv_full)


def kernel(pred_boxes, pred_confidence, pred_keypoints, target_boxes, target_keypoints):
    bsz, g = pred_boxes.shape[0], pred_boxes.shape[1]
    nkey = pred_keypoints.shape[-1]

    tb_pad = jnp.pad(target_boxes, ((0, 0), (0, NOBJ_P - NOBJ), (0, 0)))
    tvT = _run_prep(tb_pad)
    tv_full = jnp.transpose(tvT, (1, 0, 2)).reshape(16, bsz * NOBJ_P)

    box_pl = jnp.moveaxis(pred_boxes, 3, 1)
    conf_pl = pred_confidence.reshape(bsz, g, g)
    kp_pl = jnp.moveaxis(pred_keypoints, 3, 1).reshape(bsz, nkey * g, g)

    tkp_pad = jnp.pad(target_keypoints, ((0, 0), (0, NOBJ_P - NOBJ), (0, 0)))
    tkpT = jnp.transpose(tkp_pad, (0, 2, 1))
    return _run_losses(box_pl, conf_pl, kp_pl, tkpT, tvT, tv_full)


# speedup vs baseline: 5.8111x; 1.0195x over previous
import functools

import jax
import jax.numpy as jnp
from jax import lax
from jax.experimental import pallas as pl
from jax.experimental.pallas import tpu as pltpu

GRID = 128
NOBJ = 100
NOBJ_P = 128
ANCHOR_W = 4.0
ANCHOR_H = 4.0
EMPTY_WEIGHT = 5.0


def _prep_body(tb_ref, tv_ref):
    tb = tb_ref[...]
    g = jnp.float32(GRID)
    t_x = tb[..., 0] * g
    t_y = tb[..., 1] * g
    t_w = tb[..., 3] * g
    t_h = tb[..., 4] * g
    gif = jnp.floor(t_x)
    gjf = jnp.floor(t_y)
    gii = gif.astype(jnp.int32)
    gjj = gjf.astype(jnp.int32)
    bsz = tb.shape[0]
    o = lax.broadcasted_iota(jnp.int32, (bsz, NOBJ_P), 1)
    validm = o < NOBJ
    cell = jnp.where(validm, gjj * GRID + gii, -1 - o)
    eq = cell[:, :, None] == cell[:, None, :]
    later = (lax.broadcasted_iota(jnp.int32, (bsz, NOBJ_P, NOBJ_P), 2)
             > lax.broadcasted_iota(jnp.int32, (bsz, NOBJ_P, NOBJ_P), 1))
    dup = jnp.any(eq & later, axis=2)
    win = jnp.where(validm & ~dup, 1.0, 0.0).astype(jnp.float32)

    tx = t_x - gif
    ty = t_y - gjf
    tz = tb[..., 2]
    tw = jnp.log(t_w / ANCHOR_W + 1e-16)
    th = jnp.log(t_h / ANCHOR_H + 1e-16)
    td = jnp.log(tb[..., 5] + 1e-16)
    v = win > 0
    tx1 = jnp.where(v, t_x - 0.5 * t_w, 0.0)
    tx2 = jnp.where(v, t_x + 0.5 * t_w, 1.0)
    ty1 = jnp.where(v, t_y - 0.5 * t_h, 0.0)
    ty2 = jnp.where(v, t_y + 0.5 * t_h, 1.0)
    areat = jnp.where(v, t_w * t_h, 1.0)
    zero = jnp.zeros_like(tx)
    tv_ref[...] = jnp.stack(
        [tx, ty, tz, tw, th, td, win, gif, gjf,
         tx1, tx2, ty1, ty2, areat, zero, zero], axis=1)


def _run_prep(tb_pad):
    bsz = tb_pad.shape[0]
    return pl.pallas_call(
        _prep_body,
        out_shape=jax.ShapeDtypeStruct((bsz, 16, NOBJ_P), jnp.float32),
    )(tb_pad)


def _planar_select(slab2d, W_cT, W_rT, nch):
    A = jnp.dot(slab2d, W_cT, preferred_element_type=jnp.float32)
    A = A.reshape(nch, GRID, W_cT.shape[1])
    return jnp.sum(A * W_rT[None], axis=1)


def _loss_body(nsteps, ntot,
               box_ref, conf_ref, kp_ref, tkp_ref,
               tv_blk_ref, tv_full_ref, out_ref, acc_ref):
    i = pl.program_id(0)

    @pl.when(i == 0)
    def _init():
        for k in range(6):
            acc_ref[k] = 0.0

    sp = jax.nn.softplus
    tvb = tv_blk_ref[0]
    nobj = tvb.shape[1]
    w = tvb[6]
    gif = tvb[7]
    gjf = tvb[8]

    io = lax.broadcasted_iota(jnp.int32, (GRID, nobj), 0)
    W_cT = jnp.where(gif.astype(jnp.int32)[None, :] == io, 1.0, 0.0)
    W_rT = jnp.where(gjf.astype(jnp.int32)[None, :] == io, 1.0, 0.0)

    nkey = tkp_ref.shape[1]
    box_sel = _planar_select(box_ref[0].reshape(6 * GRID, GRID), W_cT, W_rT, 6)
    conf_slab = conf_ref[0]
    selc = _planar_select(conf_slab, W_cT, W_rT, 1)[0]
    kpT_sel = _planar_select(kp_ref[0], W_cT, W_rT, nkey)

    selx, sely, selz = box_sel[0], box_sel[1], box_sel[2]
    selw, selh, seld = box_sel[3], box_sel[4], box_sel[5]

    d_box = (jnp.abs(selx - tvb[0]) + jnp.abs(sely - tvb[1])
             + jnp.abs(selw - tvb[3]) + jnp.abs(selh - tvb[4])
             + jnp.abs(selz - tvb[2]) + jnp.abs(seld - tvb[5]))
    s_box = jnp.sum(d_box * w)
    s_kp = jnp.sum(jnp.sum(jnp.abs(kpT_sel - tkp_ref[0]), axis=0) * w)
    s_n = jnp.sum(w)
    s_corr = jnp.sum(w * (EMPTY_WEIGHT * sp(-selc) - sp(selc)))
    s_base = jnp.sum(sp(conf_slab))

    px = selx + gif
    py = sely + gjf
    pw = jnp.exp(selw) * ANCHOR_W
    ph = jnp.exp(selh) * ANCHOR_H
    vP = w > 0
    px1 = jnp.where(vP, px - 0.5 * pw, 0.0)[:, None]
    px2 = jnp.where(vP, px + 0.5 * pw, 1.0)[:, None]
    py1 = jnp.where(vP, py - 0.5 * ph, 0.0)[:, None]
    py2 = jnp.where(vP, py + 0.5 * ph, 1.0)[:, None]
    areap = jnp.where(vP, pw * ph, 1.0)[:, None]
    wP = w[:, None]

    tvf = tv_full_ref[...]
    tx1 = tvf[9][None, :]
    tx2 = tvf[10][None, :]
    ty1 = tvf[11][None, :]
    ty2 = tvf[12][None, :]
    areat = tvf[13][None, :]
    wTr = tvf[6][None, :]

    ix = jnp.maximum(jnp.minimum(px2, tx2) - jnp.maximum(px1, tx1), 0.0)
    iy = jnp.maximum(jnp.minimum(py2, ty2) - jnp.maximum(py1, ty1), 0.0)
    inter = ix * iy
    union = areap + areat - inter
    rowsum = jnp.sum((union - inter) / union * wTr, axis=1)
    s_iou = jnp.sum(rowsum * w)

    acc_ref[0] += s_box
    acc_ref[1] += s_kp
    acc_ref[2] += s_corr
    acc_ref[3] += s_base
    acc_ref[4] += s_iou
    acc_ref[5] += s_n

    @pl.when(i == nsteps - 1)
    def _fin():
        n = acc_ref[5]
        out_ref[...] = jnp.stack([
            acc_ref[0] / n,
            acc_ref[1] / n,
            (acc_ref[3] + acc_ref[2]) / jnp.float32(ntot),
            acc_ref[4] / n,
        ])


def _run_losses(box_pl, conf_pl, kp_pl, tkpT, tvT, tv_full):
    bsz, g = conf_pl.shape[0], conf_pl.shape[1]
    nkey = tkpT.shape[1]
    nent = tv_full.shape[1]
    nsteps = bsz
    ntot = bsz * g * g
    return pl.pallas_call(
        functools.partial(_loss_body, nsteps, ntot),
        grid=(nsteps,),
        in_specs=[
            pl.BlockSpec((1, 6, g, g), lambda i: (i, 0, 0, 0)),
            pl.BlockSpec((1, g, g), lambda i: (i, 0, 0)),
            pl.BlockSpec((1, nkey * g, g), lambda i: (i, 0, 0)),
            pl.BlockSpec((1, nkey, NOBJ_P), lambda i: (i, 0, 0)),
            pl.BlockSpec((1, 16, NOBJ_P), lambda i: (i, 0, 0)),
            pl.BlockSpec((16, nent), lambda i: (0, 0)),
        ],
        out_specs=pl.BlockSpec((4,), lambda i: (0,)),
        out_shape=jax.ShapeDtypeStruct((4,), jnp.float32),
        scratch_shapes=[pltpu.SMEM((8,), jnp.float32)],
    )(box_pl, conf_pl, kp_pl, tkpT, tvT, tv_full)


def kernel(pred_boxes, pred_confidence, pred_keypoints, target_boxes, target_keypoints):
    bsz, g = pred_boxes.shape[0], pred_boxes.shape[1]
    nkey = pred_keypoints.shape[-1]

    tb_pad = jnp.pad(target_boxes, ((0, 0), (0, NOBJ_P - NOBJ), (0, 0)))
    tvT = _run_prep(tb_pad)
    nent = ((bsz * NOBJ + 127) // 128) * 128
    tv_full = jnp.transpose(tvT[:, :, :NOBJ], (1, 0, 2)).reshape(16, bsz * NOBJ)
    tv_full = jnp.pad(tv_full, ((0, 0), (0, nent - bsz * NOBJ)))

    box_pl = jnp.moveaxis(pred_boxes, 3, 1)
    conf_pl = pred_confidence.reshape(bsz, g, g)
    kp_pl = jnp.moveaxis(pred_keypoints, 3, 1).reshape(bsz, nkey * g, g)

    tkp_pad = jnp.pad(target_keypoints, ((0, 0), (0, NOBJ_P - NOBJ), (0, 0)))
    tkpT = jnp.transpose(tkp_pad, (0, 2, 1))
    return _run_losses(box_pl, conf_pl, kp_pl, tkpT, tvT, tv_full)
